# Initial kernel scaffold; baseline (speedup 1.0000x reference)
import jax
import jax.numpy as jnp
from jax.experimental import pallas as pl


def kernel(x, edge_index, batch, c1_Wq, c1_bq, c1_Wk, c1_bk, c1_Wv, c1_bv, c1_Ws, c1_bs, c2_Wq, c2_bq, c2_Wk, c2_bk, c2_Wv, c2_bv, c2_Ws, c2_bs, c3_Wq, c3_bq, c3_Wk, c3_bk, c3_Wv, c3_bv, c3_Ws, c3_bs, l1_W, l1_b, l2_W, l2_b, l3_W, l3_b):
    raise NotImplementedError("write your pallas kernel here")


# speedup vs baseline: 1.0801x; 1.0801x over previous
import functools

import jax
import jax.numpy as jnp
from jax import lax
from jax.experimental import pallas as pl
from jax.experimental.pallas import tpu as pltpu

N_NODES = 10000
N_EDGES = 320000
D_FEAT = 128
HEADS = 4
DIM_H = 32
HC = HEADS * DIM_H
N_GRAPHS = 64

_BM = 1000


def _dense4_body(x_ref, w_ref, b_ref, q_o, k_o, v_o, s_o):
    xb = x_ref[...]
    b = b_ref[...]
    q_o[...] = jnp.dot(xb, w_ref[0], preferred_element_type=jnp.float32) + b[0:1, 0:HC]
    k_o[...] = jnp.dot(xb, w_ref[1], preferred_element_type=jnp.float32) + b[1:2, 0:HC]
    v_o[...] = jnp.dot(xb, w_ref[2], preferred_element_type=jnp.float32) + b[2:3, 0:HC]
    s_o[...] = jnp.dot(xb, w_ref[3], preferred_element_type=jnp.float32) + b[3:4, 0:HC]


def _dense4(x, Wq, bq, Wk, bk, Wv, bv, Ws, bs):
    n = x.shape[0]
    d = x.shape[1]
    grid = n // _BM
    W = jnp.stack([Wq, Wk, Wv, Ws])
    B = jnp.stack([bq, bk, bv, bs])
    bspec_x = pl.BlockSpec((_BM, d), lambda i: (i, 0))
    bspec_w = pl.BlockSpec((4, d, HC), lambda i: (0, 0, 0))
    bspec_b = pl.BlockSpec((4, HC), lambda i: (0, 0))
    bspec_o = pl.BlockSpec((_BM, HC), lambda i: (i, 0))
    out_shape = [jax.ShapeDtypeStruct((n, HC), jnp.float32)] * 4
    q, k, v, s = pl.pallas_call(
        _dense4_body,
        grid=(grid,),
        in_specs=[bspec_x, bspec_w, bspec_b],
        out_specs=[bspec_o] * 4,
        out_shape=out_shape,
    )(x, W, B)
    return q, k, v, s


def _conv(x, src, dst, Wq, bq, Wk, bk, Wv, bv, Ws, bs):
    N = x.shape[0]
    q, k, v, sk = _dense4(x, Wq, bq, Wk, bk, Wv, bv, Ws, bs)
    qh = q.reshape(N, HEADS, DIM_H)
    kh = k.reshape(N, HEADS, DIM_H)
    vh = v.reshape(N, HEADS, DIM_H)
    qi = qh[dst]
    kj = kh[src]
    vj = vh[src]
    alpha = jnp.sum(qi * kj, axis=-1) / jnp.sqrt(float(DIM_H))
    ex = jnp.exp(alpha)
    den = jax.ops.segment_sum(ex, dst, num_segments=N)
    num = jax.ops.segment_sum(ex[:, :, None] * vj, dst, num_segments=N)
    agg = num / (den + 1e-16)[:, :, None]
    return agg.reshape(N, HC) + sk


def _pool(h, batch):
    s = jax.ops.segment_sum(h, batch, num_segments=N_GRAPHS)
    cnt = jax.ops.segment_sum(jnp.ones((h.shape[0],), dtype=h.dtype), batch,
                              num_segments=N_GRAPHS)
    return s / jnp.maximum(cnt, 1.0)[:, None]


def kernel(x, edge_index, batch,
           c1_Wq, c1_bq, c1_Wk, c1_bk, c1_Wv, c1_bv, c1_Ws, c1_bs,
           c2_Wq, c2_bq, c2_Wk, c2_bk, c2_Wv, c2_bv, c2_Ws, c2_bs,
           c3_Wq, c3_bq, c3_Wk, c3_bk, c3_Wv, c3_bv, c3_Ws, c3_bs,
           l1_W, l1_b, l2_W, l2_b, l3_W, l3_b):
    src, dst = edge_index[0], edge_index[1]
    h1 = jax.nn.relu(_conv(x, src, dst, c1_Wq, c1_bq, c1_Wk, c1_bk, c1_Wv, c1_bv, c1_Ws, c1_bs))
    h2 = jax.nn.relu(_conv(h1, src, dst, c2_Wq, c2_bq, c2_Wk, c2_bk, c2_Wv, c2_bv, c2_Ws, c2_bs))
    h3 = jax.nn.relu(_conv(h2, src, dst, c3_Wq, c3_bq, c3_Wk, c3_bk, c3_Wv, c3_bv, c3_Ws, c3_bs))
    p1 = _pool(h1, batch)
    p2 = _pool(h2, batch)
    p3 = _pool(h3, batch)
    h = jnp.concatenate([p1, p2, p3], axis=1)
    h = jax.nn.relu(h @ l1_W + l1_b)
    h = jax.nn.relu(h @ l2_W + l2_b)
    h = h @ l3_W + l3_b
    return jnp.squeeze(h, axis=-1)


# speedup vs baseline: 11.0636x; 10.2429x over previous
import functools

import jax
import jax.numpy as jnp
from jax import lax
from jax.experimental import pallas as pl
from jax.experimental.pallas import tpu as pltpu
from jax.experimental.pallas import tpu_sc as plsc

N_NODES = 10000
N_EDGES = 320000
D_FEAT = 128
HEADS = 4
DIM_H = 32
HC = HEADS * DIM_H
N_GRAPHS = 64

_BM = 1000
_BE = 2000

NC, NS = 2, 16
EPW = N_EDGES // (NC * NS)
CHUNK = 80
NCHUNK = EPW // CHUNK
N_PAD = 10112
ROWS_PT = N_PAD // NS


def _dense4_body(x_ref, w_ref, b_ref, q_o, k_o, v_o, s_o):
    xb = x_ref[...]
    b = b_ref[...]
    scale = 1.0 / (DIM_H ** 0.5)
    q = jnp.dot(xb, w_ref[0], preferred_element_type=jnp.float32) + b[0:1, 0:HC]
    q_o[...] = q * scale
    k_o[...] = jnp.dot(xb, w_ref[1], preferred_element_type=jnp.float32) + b[1:2, 0:HC]
    v_o[...] = jnp.dot(xb, w_ref[2], preferred_element_type=jnp.float32) + b[2:3, 0:HC]
    s_o[...] = jnp.dot(xb, w_ref[3], preferred_element_type=jnp.float32) + b[3:4, 0:HC]


def _dense4(x, Wq, bq, Wk, bk, Wv, bv, Ws, bs):
    n = x.shape[0]
    d = x.shape[1]
    grid = n // _BM
    W = jnp.stack([Wq, Wk, Wv, Ws])
    B = jnp.stack([bq, bk, bv, bs])
    bspec_x = pl.BlockSpec((_BM, d), lambda i: (i, 0))
    bspec_w = pl.BlockSpec((4, d, HC), lambda i: (0, 0, 0))
    bspec_b = pl.BlockSpec((4, HC), lambda i: (0, 0))
    bspec_o = pl.BlockSpec((_BM, HC), lambda i: (i, 0))
    out_shape = [jax.ShapeDtypeStruct((n, HC), jnp.float32)] * 4
    q, k, v, s = pl.pallas_call(
        _dense4_body,
        grid=(grid,),
        in_specs=[bspec_x, bspec_w, bspec_b],
        out_specs=[bspec_o] * 4,
        out_shape=out_shape,
    )(x, W, B)
    return q, k, v, s


def _gather_body(q_hbm, k_hbm, v_hbm, src_hbm, dst_hbm,
                 qd_out, ks_out, vs_out,
                 sidx, didx, qr, kr, vr, sem):
    c = lax.axis_index("c")
    s = lax.axis_index("s")
    base = (c * NS + s) * EPW

    def chunk_body(ci, carry):
        off = base + ci * CHUNK
        pltpu.sync_copy(src_hbm.at[pl.ds(off, CHUNK)], sidx)
        pltpu.sync_copy(dst_hbm.at[pl.ds(off, CHUNK)], didx)
        pltpu.async_copy(q_hbm.at[didx], qr, sem).wait()
        pltpu.async_copy(k_hbm.at[sidx], kr, sem).wait()
        pltpu.async_copy(v_hbm.at[sidx], vr, sem).wait()
        pltpu.sync_copy(qr, qd_out.at[pl.ds(off, CHUNK)])
        pltpu.sync_copy(kr, ks_out.at[pl.ds(off, CHUNK)])
        pltpu.sync_copy(vr, vs_out.at[pl.ds(off, CHUNK)])
        return carry

    lax.fori_loop(0, NCHUNK, chunk_body, 0)


def _edge_gather(q, k, v, src, dst):
    mesh = plsc.VectorSubcoreMesh(core_axis_name="c", subcore_axis_name="s")
    f = pl.kernel(
        _gather_body,
        out_type=[jax.ShapeDtypeStruct((N_EDGES, HC), jnp.float32)] * 3,
        mesh=mesh,
        scratch_types=[
            pltpu.VMEM((CHUNK,), jnp.int32),
            pltpu.VMEM((CHUNK,), jnp.int32),
            pltpu.VMEM((CHUNK, HC), jnp.float32),
            pltpu.VMEM((CHUNK, HC), jnp.float32),
            pltpu.VMEM((CHUNK, HC), jnp.float32),
            pltpu.SemaphoreType.DMA,
        ],
    )
    return f(q, k, v, src, dst)


def _edge_math_body(qd_ref, ks_ref, vs_ref, w_o, dw_o):
    qd = qd_ref[...]
    ks = ks_ref[...]
    vs = vs_ref[...]
    exs = []
    for h in range(HEADS):
        o = DIM_H * h
        a = jnp.sum(qd[:, o:o + DIM_H] * ks[:, o:o + DIM_H], axis=1,
                    keepdims=True)
        ex = jnp.exp(a)
        exs.append(ex)
        w_o[:, o:o + DIM_H] = vs[:, o:o + DIM_H] * ex
    dw_o[...] = jnp.concatenate(
        exs + [jnp.zeros((qd.shape[0], 16 - HEADS), jnp.float32)], axis=1)


def _edge_math(qd, ks, vs):
    grid = N_EDGES // _BE
    bspec_in = pl.BlockSpec((_BE, HC), lambda i: (i, 0))
    bspec_w = pl.BlockSpec((_BE, HC), lambda i: (i, 0))
    bspec_dw = pl.BlockSpec((_BE, 16), lambda i: (i, 0))
    w, dw = pl.pallas_call(
        _edge_math_body,
        grid=(grid,),
        in_specs=[bspec_in] * 3,
        out_specs=[bspec_w, bspec_dw],
        out_shape=[
            jax.ShapeDtypeStruct((N_EDGES, HC), jnp.float32),
            jax.ShapeDtypeStruct((N_EDGES, 16), jnp.float32),
        ],
    )(qd, ks, vs)
    return w, dw


def _scatter_body(w_hbm, dw_hbm, dst_hbm, zn_hbm, zd_hbm,
                  num_out, den_out,
                  didx, wr, dwr, num_sh, den_sh, sem):
    c = lax.axis_index("c")
    s = lax.axis_index("s")

    r0 = s * ROWS_PT
    pltpu.sync_copy(zn_hbm.at[pl.ds(r0, ROWS_PT)], num_sh.at[pl.ds(r0, ROWS_PT)])
    pltpu.sync_copy(zd_hbm.at[pl.ds(r0, ROWS_PT)], den_sh.at[pl.ds(r0, ROWS_PT)])
    plsc.subcore_barrier()

    base = (c * NS + s) * EPW

    def chunk_body(ci, carry):
        off = base + ci * CHUNK
        pltpu.sync_copy(dst_hbm.at[pl.ds(off, CHUNK)], didx)
        pltpu.sync_copy(w_hbm.at[pl.ds(off, CHUNK)], wr)
        pltpu.sync_copy(dw_hbm.at[pl.ds(off, CHUNK)], dwr)
        pltpu.sync_copy(wr, num_sh.at[didx], add=True)
        pltpu.sync_copy(dwr, den_sh.at[didx], add=True)
        return carry

    lax.fori_loop(0, NCHUNK, chunk_body, 0)
    plsc.subcore_barrier()

    pltpu.sync_copy(num_sh.at[pl.ds(r0, ROWS_PT)], num_out.at[c, pl.ds(r0, ROWS_PT)])
    pltpu.sync_copy(den_sh.at[pl.ds(r0, ROWS_PT)], den_out.at[c, pl.ds(r0, ROWS_PT)])


def _edge_scatter(w, dw, dst):
    zn = jnp.zeros((N_PAD, HC), jnp.float32)
    zd = jnp.zeros((N_PAD, 16), jnp.float32)
    mesh = plsc.VectorSubcoreMesh(core_axis_name="c", subcore_axis_name="s")
    f = pl.kernel(
        _scatter_body,
        out_type=[
            jax.ShapeDtypeStruct((NC, N_PAD, HC), jnp.float32),
            jax.ShapeDtypeStruct((NC, N_PAD, 16), jnp.float32),
        ],
        mesh=mesh,
        scratch_types=[
            pltpu.VMEM((CHUNK,), jnp.int32),
            pltpu.VMEM((CHUNK, HC), jnp.float32),
            pltpu.VMEM((CHUNK, 16), jnp.float32),
            pltpu.MemorySpace.VMEM_SHARED((N_PAD, HC), jnp.float32),
            pltpu.MemorySpace.VMEM_SHARED((N_PAD, 16), jnp.float32),
            pltpu.SemaphoreType.DMA,
        ],
    )
    num2, den2 = f(w, dw, dst, zn, zd)
    return (num2[0, :N_NODES] + num2[1, :N_NODES],
            den2[0, :N_NODES] + den2[1, :N_NODES])


def _conv(x, src, dst, Wq, bq, Wk, bk, Wv, bv, Ws, bs):
    N = x.shape[0]
    q, k, v, sk = _dense4(x, Wq, bq, Wk, bk, Wv, bv, Ws, bs)
    qd, ks, vs = _edge_gather(q, k, v, src, dst)
    w, dw = _edge_math(qd, ks, vs)
    num = jax.ops.segment_sum(w, dst, num_segments=N)
    den = jax.ops.segment_sum(dw, dst, num_segments=N)
    den_h = den[:, :HEADS]
    agg = num.reshape(N, HEADS, DIM_H) / (den_h[:, :, None] + 1e-16)
    return agg.reshape(N, HC) + sk


def _pool(h, batch):
    s = jax.ops.segment_sum(h, batch, num_segments=N_GRAPHS)
    cnt = jax.ops.segment_sum(jnp.ones((h.shape[0],), dtype=h.dtype), batch,
                              num_segments=N_GRAPHS)
    return s / jnp.maximum(cnt, 1.0)[:, None]


def kernel(x, edge_index, batch,
           c1_Wq, c1_bq, c1_Wk, c1_bk, c1_Wv, c1_bv, c1_Ws, c1_bs,
           c2_Wq, c2_bq, c2_Wk, c2_bk, c2_Wv, c2_bv, c2_Ws, c2_bs,
           c3_Wq, c3_bq, c3_Wk, c3_bk, c3_Wv, c3_bv, c3_Ws, c3_bs,
           l1_W, l1_b, l2_W, l2_b, l3_W, l3_b):
    src, dst = edge_index[0], edge_index[1]
    h1 = jax.nn.relu(_conv(x, src, dst, c1_Wq, c1_bq, c1_Wk, c1_bk, c1_Wv, c1_bv, c1_Ws, c1_bs))
    h2 = jax.nn.relu(_conv(h1, src, dst, c2_Wq, c2_bq, c2_Wk, c2_bk, c2_Wv, c2_bv, c2_Ws, c2_bs))
    h3 = jax.nn.relu(_conv(h2, src, dst, c3_Wq, c3_bq, c3_Wk, c3_bk, c3_Wv, c3_bv, c3_Ws, c3_bs))
    p1 = _pool(h1, batch)
    p2 = _pool(h2, batch)
    p3 = _pool(h3, batch)
    h = jnp.concatenate([p1, p2, p3], axis=1)
    h = jax.nn.relu(h @ l1_W + l1_b)
    h = jax.nn.relu(h @ l2_W + l2_b)
    h = h @ l3_W + l3_b
    return jnp.squeeze(h, axis=-1)


# speedup vs baseline: 11.0753x; 1.0011x over previous
import jax
import jax.numpy as jnp
from jax import lax
from jax.experimental import pallas as pl
from jax.experimental.pallas import tpu as pltpu
from jax.experimental.pallas import tpu_sc as plsc

N_NODES = 10000
N_EDGES = 320000
D_FEAT = 128
HEADS = 4
DIM_H = 32
HC = HEADS * DIM_H
N_GRAPHS = 64

_BM = 1000
_BE = 2000

NC, NS = 2, 16
EPW = N_EDGES // (NC * NS)
CHUNK = 80
NCHUNK = EPW // CHUNK


def _dense4_body(x_ref, w_ref, b_ref, q_o, k_o, v_o, s_o):
    xb = x_ref[...]
    b = b_ref[...]
    scale = 1.0 / (DIM_H ** 0.5)
    q = jnp.dot(xb, w_ref[0], preferred_element_type=jnp.float32) + b[0:1, 0:HC]
    q_o[...] = q * scale
    k_o[...] = jnp.dot(xb, w_ref[1], preferred_element_type=jnp.float32) + b[1:2, 0:HC]
    v_o[...] = jnp.dot(xb, w_ref[2], preferred_element_type=jnp.float32) + b[2:3, 0:HC]
    s_o[...] = jnp.dot(xb, w_ref[3], preferred_element_type=jnp.float32) + b[3:4, 0:HC]


def _dense4(x, Wq, bq, Wk, bk, Wv, bv, Ws, bs):
    n = x.shape[0]
    d = x.shape[1]
    grid = n // _BM
    W = jnp.stack([Wq, Wk, Wv, Ws])
    B = jnp.stack([bq, bk, bv, bs])
    bspec_x = pl.BlockSpec((_BM, d), lambda i: (i, 0))
    bspec_w = pl.BlockSpec((4, d, HC), lambda i: (0, 0, 0))
    bspec_b = pl.BlockSpec((4, HC), lambda i: (0, 0))
    bspec_o = pl.BlockSpec((_BM, HC), lambda i: (i, 0))
    out_shape = [jax.ShapeDtypeStruct((n, HC), jnp.float32)] * 4
    q, k, v, s = pl.pallas_call(
        _dense4_body,
        grid=(grid,),
        in_specs=[bspec_x, bspec_w, bspec_b],
        out_specs=[bspec_o] * 4,
        out_shape=out_shape,
    )(x, W, B)
    return q, k, v, s


def _gather_body(q_hbm, k_hbm, v_hbm, src_hbm, dst_hbm,
                 qd_out, ks_out, vs_out,
                 sidx, didx, qr, kr, vr, sem):
    c = lax.axis_index("c")
    s = lax.axis_index("s")
    base = (c * NS + s) * EPW

    def chunk_body(ci, carry):
        off = base + ci * CHUNK
        pltpu.sync_copy(src_hbm.at[pl.ds(off, CHUNK)], sidx)
        pltpu.sync_copy(dst_hbm.at[pl.ds(off, CHUNK)], didx)
        pltpu.async_copy(q_hbm.at[didx], qr, sem).wait()
        pltpu.async_copy(k_hbm.at[sidx], kr, sem).wait()
        pltpu.async_copy(v_hbm.at[sidx], vr, sem).wait()
        pltpu.sync_copy(qr, qd_out.at[pl.ds(off, CHUNK)])
        pltpu.sync_copy(kr, ks_out.at[pl.ds(off, CHUNK)])
        pltpu.sync_copy(vr, vs_out.at[pl.ds(off, CHUNK)])
        return carry

    lax.fori_loop(0, NCHUNK, chunk_body, 0)


def _edge_gather(q, k, v, src, dst):
    mesh = plsc.VectorSubcoreMesh(core_axis_name="c", subcore_axis_name="s")
    f = pl.kernel(
        _gather_body,
        out_type=[jax.ShapeDtypeStruct((N_EDGES, HC), jnp.float32)] * 3,
        mesh=mesh,
        scratch_types=[
            pltpu.VMEM((CHUNK,), jnp.int32),
            pltpu.VMEM((CHUNK,), jnp.int32),
            pltpu.VMEM((CHUNK, HC), jnp.float32),
            pltpu.VMEM((CHUNK, HC), jnp.float32),
            pltpu.VMEM((CHUNK, HC), jnp.float32),
            pltpu.SemaphoreType.DMA,
        ],
    )
    return f(q, k, v, src, dst)


def _edge_math_body(qd_ref, ks_ref, vs_ref, w_o, dw_o):
    qd = qd_ref[...]
    ks = ks_ref[...]
    vs = vs_ref[...]
    exs = []
    for h in range(HEADS):
        o = DIM_H * h
        a = jnp.sum(qd[:, o:o + DIM_H] * ks[:, o:o + DIM_H], axis=1,
                    keepdims=True)
        ex = jnp.exp(a)
        exs.append(ex)
        w_o[:, o:o + DIM_H] = vs[:, o:o + DIM_H] * ex
    dw_o[...] = jnp.concatenate(
        exs + [jnp.zeros((qd.shape[0], 16 - HEADS), jnp.float32)], axis=1)


def _edge_math(qd, ks, vs):
    grid = N_EDGES // _BE
    bspec_in = pl.BlockSpec((_BE, HC), lambda i: (i, 0))
    bspec_w = pl.BlockSpec((_BE, HC), lambda i: (i, 0))
    bspec_dw = pl.BlockSpec((_BE, 16), lambda i: (i, 0))
    w, dw = pl.pallas_call(
        _edge_math_body,
        grid=(grid,),
        in_specs=[bspec_in] * 3,
        out_specs=[bspec_w, bspec_dw],
        out_shape=[
            jax.ShapeDtypeStruct((N_EDGES, HC), jnp.float32),
            jax.ShapeDtypeStruct((N_EDGES, 16), jnp.float32),
        ],
    )(qd, ks, vs)
    return w, dw


def _conv(x, src, dst, Wq, bq, Wk, bk, Wv, bv, Ws, bs):
    N = x.shape[0]
    q, k, v, sk = _dense4(x, Wq, bq, Wk, bk, Wv, bv, Ws, bs)
    qd, ks, vs = _edge_gather(q, k, v, src, dst)
    w, dw = _edge_math(qd, ks, vs)
    num = jax.ops.segment_sum(w, dst, num_segments=N)
    den = jax.ops.segment_sum(dw, dst, num_segments=N)
    den_h = den[:, :HEADS]
    agg = num.reshape(N, HEADS, DIM_H) / (den_h[:, :, None] + 1e-16)
    return agg.reshape(N, HC) + sk


def _pool(h, batch):
    s = jax.ops.segment_sum(h, batch, num_segments=N_GRAPHS)
    cnt = jax.ops.segment_sum(jnp.ones((h.shape[0],), dtype=h.dtype), batch,
                              num_segments=N_GRAPHS)
    return s / jnp.maximum(cnt, 1.0)[:, None]


def kernel(x, edge_index, batch,
           c1_Wq, c1_bq, c1_Wk, c1_bk, c1_Wv, c1_bv, c1_Ws, c1_bs,
           c2_Wq, c2_bq, c2_Wk, c2_bk, c2_Wv, c2_bv, c2_Ws, c2_bs,
           c3_Wq, c3_bq, c3_Wk, c3_bk, c3_Wv, c3_bv, c3_Ws, c3_bs,
           l1_W, l1_b, l2_W, l2_b, l3_W, l3_b):
    src, dst = edge_index[0], edge_index[1]
    h1 = jax.nn.relu(_conv(x, src, dst, c1_Wq, c1_bq, c1_Wk, c1_bk, c1_Wv, c1_bv, c1_Ws, c1_bs))
    h2 = jax.nn.relu(_conv(h1, src, dst, c2_Wq, c2_bq, c2_Wk, c2_bk, c2_Wv, c2_bv, c2_Ws, c2_bs))
    h3 = jax.nn.relu(_conv(h2, src, dst, c3_Wq, c3_bq, c3_Wk, c3_bk, c3_Wv, c3_bv, c3_Ws, c3_bs))
    p1 = _pool(h1, batch)
    p2 = _pool(h2, batch)
    p3 = _pool(h3, batch)
    h = jnp.concatenate([p1, p2, p3], axis=1)
    h = jax.nn.relu(h @ l1_W + l1_b)
    h = jax.nn.relu(h @ l2_W + l2_b)
    h = h @ l3_W + l3_b
    return jnp.squeeze(h, axis=-1)
